# baseline (speedup 1.0000x reference)
import jax
import jax.numpy as jnp
from jax.experimental import pallas as pl

_BLK = 4096


def _copy_body(in_ref, out_ref):
    out_ref[...] = in_ref[...]


def kernel(queue):
    rows, dim = queue.shape
    return pl.pallas_call(
        _copy_body,
        out_shape=jax.ShapeDtypeStruct(queue.shape, queue.dtype),
        grid=(rows // _BLK,),
        in_specs=[pl.BlockSpec((_BLK, dim), lambda i: (i, 0))],
        out_specs=pl.BlockSpec((_BLK, dim), lambda i: (i, 0)),
    )(queue)
